# baseline (speedup 1.0000x reference)
import functools

import jax
import jax.numpy as jnp
from jax import lax
from jax.experimental import pallas as pl
from jax.experimental.pallas import tpu as pltpu
from jax.experimental.pallas import tpu_sc as plsc

_B, _N, _Ci, _Cp, _H, _W = 2, 16384, 64, 96, 192, 640
_M = _B * _N
_TM = 1024
_CH = 128
_EPS = 1e-5


def _idx_body(xn_ref, yn_ref, idx_ref, wgt_ref):
    xn = xn_ref[...]
    yn = yn_ref[...]
    x = (xn + 1.0) * (_W / 2.0) - 0.5
    y = (yn + 1.0) * (_H / 2.0) - 0.5
    x0 = jnp.floor(x)
    y0 = jnp.floor(y)
    wx1 = x - x0
    wx0 = 1.0 - wx1
    wy1 = y - y0
    wy0 = 1.0 - wy1
    rows = xn.shape[0]
    boff = (lax.broadcasted_iota(jnp.int32, (rows, 128), 0) // (_N // 128)) * (_H * _W)

    def corner(c, xi, yi, wgt):
        valid = ((xi >= 0) & (xi <= _W - 1) & (yi >= 0) & (yi <= _H - 1))
        xi_c = jnp.clip(xi, 0.0, _W - 1).astype(jnp.int32)
        yi_c = jnp.clip(yi, 0.0, _H - 1).astype(jnp.int32)
        pix = yi_c * _W + xi_c + boff
        w = wgt * valid.astype(jnp.float32)
        par = (pix & 1).astype(jnp.float32)
        idx_ref[c] = pix >> 1
        wgt_ref[2 * c] = w * (1.0 - par)
        wgt_ref[2 * c + 1] = w * par

    corner(0, x0, y0, wx0 * wy0)
    corner(1, x0 + 1.0, y0, wx1 * wy0)
    corner(2, x0, y0 + 1.0, wx0 * wy1)
    corner(3, x0 + 1.0, y0 + 1.0, wx1 * wy1)


def _compute_idx(xn, yn):
    rows = _M // 128
    return pl.pallas_call(
        _idx_body,
        out_shape=(
            jax.ShapeDtypeStruct((4, rows, 128), jnp.int32),
            jax.ShapeDtypeStruct((8, rows, 128), jnp.float32),
        ),
    )(xn.reshape(rows, 128), yn.reshape(rows, 128))


def _sc_gather(table, idx):
    total = 4 * _M
    info = plsc.get_sparse_core_info()
    nw = info.num_cores * info.num_subcores
    per_w = total // nw
    n_chunks = per_w // _CH
    mesh = plsc.VectorSubcoreMesh(core_axis_name="c", subcore_axis_name="s")

    @functools.partial(
        pl.kernel,
        mesh=mesh,
        out_type=jax.ShapeDtypeStruct((total, 128), jnp.float32),
        scratch_types=[
            pltpu.VMEM((_CH,), jnp.int32),
            pltpu.VMEM((_CH, 128), jnp.float32),
            pltpu.SemaphoreType.DMA,
        ],
    )
    def k(table_hbm, idx_hbm, out_hbm, idx_v, rows_v, sem):
        wid = lax.axis_index("s") * info.num_cores + lax.axis_index("c")
        base = wid * per_w

        def body(i, carry):
            off = base + i * _CH
            pltpu.sync_copy(idx_hbm.at[pl.ds(off, _CH)], idx_v)
            pltpu.async_copy(table_hbm.at[idx_v], rows_v, sem).wait()
            pltpu.sync_copy(rows_v, out_hbm.at[pl.ds(off, _CH)])
            return carry

        lax.fori_loop(0, n_chunks, body, 0)

    return k(table, idx)


def _dense_a_body(v4_ref, w_ref, pf_ref, Wfc1_ref, bfc1_ref, Wfc2_ref, bfc2_ref,
                  Wfc3_ref, bfc3_ref, WconvT_ref, bconv_ref,
                  imgn_ref, att_ref, s1_ref, s2_ref):
    v4 = v4_ref[...]
    w = w_ref[...]
    img = (v4[0, :, :_Ci] * w[:, 0:1] + v4[0, :, _Ci:] * w[:, 1:2]
           + v4[1, :, :_Ci] * w[:, 2:3] + v4[1, :, _Ci:] * w[:, 3:4]
           + v4[2, :, :_Ci] * w[:, 4:5] + v4[2, :, _Ci:] * w[:, 5:6]
           + v4[3, :, :_Ci] * w[:, 6:7] + v4[3, :, _Ci:] * w[:, 7:8])
    pf = pf_ref[...]
    ri = jnp.dot(img, Wfc1_ref[...], preferred_element_type=jnp.float32) + bfc1_ref[...]
    rp = jnp.dot(pf, Wfc2_ref[...], preferred_element_type=jnp.float32) + bfc2_ref[...]
    t = jnp.tanh(ri + rp)
    att = jax.nn.sigmoid(
        jnp.dot(t, Wfc3_ref[...], preferred_element_type=jnp.float32) + bfc3_ref[...])
    imgn = jnp.dot(img, WconvT_ref[...], preferred_element_type=jnp.float32) + bconv_ref[...]
    imgn_ref[...] = imgn
    att_ref[...] = att

    @pl.when(pl.program_id(0) == 0)
    def _():
        s1_ref[...] = jnp.zeros_like(s1_ref)
        s2_ref[...] = jnp.zeros_like(s2_ref)

    s1_ref[...] += jnp.sum(imgn, axis=0, keepdims=True)
    s2_ref[...] += jnp.sum(imgn * imgn, axis=0, keepdims=True)


def _dense_a(v4, wgt, pf, Wfc1, bfc1, Wfc2, bfc2, Wfc3, bfc3, WconvT, bconv):
    grid = _M // _TM
    return pl.pallas_call(
        _dense_a_body,
        grid=(grid,),
        in_specs=[
            pl.BlockSpec((4, _TM, 128), lambda i: (0, i, 0)),
            pl.BlockSpec((_TM, 8), lambda i: (i, 0)),
            pl.BlockSpec((_TM, _Cp), lambda i: (i, 0)),
            pl.BlockSpec((_Ci, 24), lambda i: (0, 0)),
            pl.BlockSpec((1, 24), lambda i: (0, 0)),
            pl.BlockSpec((_Cp, 24), lambda i: (0, 0)),
            pl.BlockSpec((1, 24), lambda i: (0, 0)),
            pl.BlockSpec((24, 1), lambda i: (0, 0)),
            pl.BlockSpec((1, 1), lambda i: (0, 0)),
            pl.BlockSpec((_Ci, _Cp), lambda i: (0, 0)),
            pl.BlockSpec((1, _Cp), lambda i: (0, 0)),
        ],
        out_specs=[
            pl.BlockSpec((_TM, _Cp), lambda i: (i, 0)),
            pl.BlockSpec((_TM, 1), lambda i: (i, 0)),
            pl.BlockSpec((1, _Cp), lambda i: (0, 0)),
            pl.BlockSpec((1, _Cp), lambda i: (0, 0)),
        ],
        out_shape=[
            jax.ShapeDtypeStruct((_M, _Cp), jnp.float32),
            jax.ShapeDtypeStruct((_M, 1), jnp.float32),
            jax.ShapeDtypeStruct((1, _Cp), jnp.float32),
            jax.ShapeDtypeStruct((1, _Cp), jnp.float32),
        ],
        compiler_params=pltpu.CompilerParams(
            dimension_semantics=("arbitrary",)),
    )(v4, wgt, pf, Wfc1, bfc1, Wfc2, bfc2, Wfc3, bfc3, WconvT, bconv)


def _dense_b_body(imgn_ref, att_ref, pf_ref, s1_ref, s2_ref, g1_ref, be1_ref,
                  WfpT_ref, WfiT_ref, bfuse_ref, h_ref, t1_ref, t2_ref):
    s1 = s1_ref[...]
    s2 = s2_ref[...]
    mean = s1 * (1.0 / _M)
    var = s2 * (1.0 / _M) - mean * mean
    scale = g1_ref[...] * jax.lax.rsqrt(var + _EPS)
    shift = be1_ref[...] - mean * scale
    img_out = jnp.maximum(imgn_ref[...] * scale + shift, 0.0) * att_ref[...]
    h = (jnp.dot(pf_ref[...], WfpT_ref[...], preferred_element_type=jnp.float32)
         + jnp.dot(img_out, WfiT_ref[...], preferred_element_type=jnp.float32)
         + bfuse_ref[...])
    h_ref[...] = h

    @pl.when(pl.program_id(0) == 0)
    def _():
        t1_ref[...] = jnp.zeros_like(t1_ref)
        t2_ref[...] = jnp.zeros_like(t2_ref)

    t1_ref[...] += jnp.sum(h, axis=0, keepdims=True)
    t2_ref[...] += jnp.sum(h * h, axis=0, keepdims=True)


def _dense_b(imgn, att, pf, s1, s2, g1, be1, WfpT, WfiT, bfuse):
    grid = _M // _TM
    return pl.pallas_call(
        _dense_b_body,
        grid=(grid,),
        in_specs=[
            pl.BlockSpec((_TM, _Cp), lambda i: (i, 0)),
            pl.BlockSpec((_TM, 1), lambda i: (i, 0)),
            pl.BlockSpec((_TM, _Cp), lambda i: (i, 0)),
            pl.BlockSpec((1, _Cp), lambda i: (0, 0)),
            pl.BlockSpec((1, _Cp), lambda i: (0, 0)),
            pl.BlockSpec((1, _Cp), lambda i: (0, 0)),
            pl.BlockSpec((1, _Cp), lambda i: (0, 0)),
            pl.BlockSpec((_Cp, _Cp), lambda i: (0, 0)),
            pl.BlockSpec((_Cp, _Cp), lambda i: (0, 0)),
            pl.BlockSpec((1, _Cp), lambda i: (0, 0)),
        ],
        out_specs=[
            pl.BlockSpec((_TM, _Cp), lambda i: (i, 0)),
            pl.BlockSpec((1, _Cp), lambda i: (0, 0)),
            pl.BlockSpec((1, _Cp), lambda i: (0, 0)),
        ],
        out_shape=[
            jax.ShapeDtypeStruct((_M, _Cp), jnp.float32),
            jax.ShapeDtypeStruct((1, _Cp), jnp.float32),
            jax.ShapeDtypeStruct((1, _Cp), jnp.float32),
        ],
        compiler_params=pltpu.CompilerParams(
            dimension_semantics=("arbitrary",)),
    )(imgn, att, pf, s1, s2, g1, be1, WfpT, WfiT, bfuse)


def _dense_c_body(h_ref, t1_ref, t2_ref, g2_ref, be2_ref, o_ref):
    t1 = t1_ref[...]
    t2 = t2_ref[...]
    mean = t1 * (1.0 / _M)
    var = t2 * (1.0 / _M) - mean * mean
    scale = g2_ref[...] * jax.lax.rsqrt(var + _EPS)
    shift = be2_ref[...] - mean * scale
    o_ref[...] = jnp.maximum(h_ref[...] * scale + shift, 0.0)


def _dense_c(h, t1, t2, g2, be2):
    grid = _M // _TM
    return pl.pallas_call(
        _dense_c_body,
        grid=(grid,),
        in_specs=[
            pl.BlockSpec((_TM, _Cp), lambda i: (i, 0)),
            pl.BlockSpec((1, _Cp), lambda i: (0, 0)),
            pl.BlockSpec((1, _Cp), lambda i: (0, 0)),
            pl.BlockSpec((1, _Cp), lambda i: (0, 0)),
            pl.BlockSpec((1, _Cp), lambda i: (0, 0)),
        ],
        out_specs=pl.BlockSpec((_TM, _Cp), lambda i: (i, 0)),
        out_shape=jax.ShapeDtypeStruct((_M, _Cp), jnp.float32),
        compiler_params=pltpu.CompilerParams(
            dimension_semantics=("arbitrary",)),
    )(h, t1, t2, g2, be2)


@jax.jit
def kernel(point_features, image, xy, Wfc1, bfc1, Wfc2, bfc2, Wfc3, bfc3,
           Wconv, bconv, g1, be1, Wfuse, bfuse, g2, be2):
    table = image.transpose(0, 2, 3, 1).reshape(_B * _H * _W // 2, 2 * _Ci)
    pf = point_features.transpose(0, 2, 1).reshape(_M, _Cp)
    xn = xy[..., 0].reshape(_M)
    yn = xy[..., 1].reshape(_M)

    idx, wgt = _compute_idx(xn, yn)
    gathered = _sc_gather(table, idx.reshape(4 * _M))
    v4 = gathered.reshape(4, _M, 128)
    wgt_m = wgt.reshape(8, _M).T

    imgn, att, s1, s2 = _dense_a(
        v4, wgt_m, pf,
        Wfc1, bfc1.reshape(1, 24), Wfc2, bfc2.reshape(1, 24),
        Wfc3, bfc3.reshape(1, 1),
        Wconv.T, bconv.reshape(1, _Cp))

    h, t1, t2 = _dense_b(
        imgn, att, pf, s1, s2,
        g1.reshape(1, _Cp), be1.reshape(1, _Cp),
        Wfuse[:, :_Cp].T, Wfuse[:, _Cp:].T, bfuse.reshape(1, _Cp))

    out = _dense_c(h, t1, t2, g2.reshape(1, _Cp), be2.reshape(1, _Cp))
    return out.reshape(_B, _N, _Cp).transpose(0, 2, 1)


# speedup vs baseline: 1.1558x; 1.1558x over previous
import functools

import jax
import jax.numpy as jnp
from jax import lax
from jax.experimental import pallas as pl
from jax.experimental.pallas import tpu as pltpu
from jax.experimental.pallas import tpu_sc as plsc

_B, _N, _Ci, _Cp, _H, _W = 2, 16384, 64, 96, 192, 640
_M = _B * _N
_TM = 1024
_CH = 128
_EPS = 1e-5


def _idx_body(xn_ref, yn_ref, idx_ref, wgt_ref):
    xn = xn_ref[...]
    yn = yn_ref[...]
    x = (xn + 1.0) * (_W / 2.0) - 0.5
    y = (yn + 1.0) * (_H / 2.0) - 0.5
    x0 = jnp.floor(x)
    y0 = jnp.floor(y)
    wx1 = x - x0
    wx0 = 1.0 - wx1
    wy1 = y - y0
    wy0 = 1.0 - wy1
    rows = xn.shape[0]
    boff = (lax.broadcasted_iota(jnp.int32, (rows, 128), 0) // (_N // 128)) * (_H * _W)

    def corner(c, xi, yi, wgt):
        valid = ((xi >= 0) & (xi <= _W - 1) & (yi >= 0) & (yi <= _H - 1))
        xi_c = jnp.clip(xi, 0.0, _W - 1).astype(jnp.int32)
        yi_c = jnp.clip(yi, 0.0, _H - 1).astype(jnp.int32)
        pix = yi_c * _W + xi_c + boff
        w = wgt * valid.astype(jnp.float32)
        par = (pix & 1).astype(jnp.float32)
        idx_ref[c] = pix >> 1
        wgt_ref[2 * c] = w * (1.0 - par)
        wgt_ref[2 * c + 1] = w * par

    corner(0, x0, y0, wx0 * wy0)
    corner(1, x0 + 1.0, y0, wx1 * wy0)
    corner(2, x0, y0 + 1.0, wx0 * wy1)
    corner(3, x0 + 1.0, y0 + 1.0, wx1 * wy1)


def _compute_idx(xn, yn):
    rows = _M // 128
    return pl.pallas_call(
        _idx_body,
        out_shape=(
            jax.ShapeDtypeStruct((4, rows, 128), jnp.int32),
            jax.ShapeDtypeStruct((8, rows, 128), jnp.float32),
        ),
    )(xn.reshape(rows, 128), yn.reshape(rows, 128))


def _sc_gather(table, idx):
    total = 4 * _M
    info = plsc.get_sparse_core_info()
    nw = info.num_cores * info.num_subcores
    per_w = total // nw
    n_chunks = per_w // _CH
    mesh = plsc.VectorSubcoreMesh(core_axis_name="c", subcore_axis_name="s")

    nb = 4
    n_groups = n_chunks // nb

    @functools.partial(
        pl.kernel,
        mesh=mesh,
        out_type=jax.ShapeDtypeStruct((total, 128), jnp.float32),
        scratch_types=(
            [pltpu.VMEM((per_w,), jnp.int32)]
            + [pltpu.VMEM((_CH, 128), jnp.float32) for _ in range(nb)]
            + [pltpu.SemaphoreType.DMA, pltpu.SemaphoreType.DMA]
        ),
    )
    def k(table_hbm, idx_hbm, out_hbm, idx_v, b0, b1, b2, b3, gsem, ssem):
        bufs = (b0, b1, b2, b3)
        wid = lax.axis_index("s") * info.num_cores + lax.axis_index("c")
        base = wid * per_w
        pltpu.sync_copy(idx_hbm.at[pl.ds(base, per_w)], idx_v)

        def body(g, carry):
            off0 = g * (nb * _CH)
            copies = []
            for b in range(nb):
                idx_slice = idx_v.at[pl.ds(off0 + b * _CH, _CH)]
                copies.append(pltpu.async_copy(table_hbm.at[idx_slice], bufs[b], gsem))
            for c in copies:
                c.wait()
            stores = []
            for b in range(nb):
                dst = out_hbm.at[pl.ds(base + off0 + b * _CH, _CH)]
                stores.append(pltpu.async_copy(bufs[b], dst, ssem))
            for s in stores:
                s.wait()
            return carry

        lax.fori_loop(0, n_groups, body, 0)

    return k(table, idx)


def _dense_a_body(v4_ref, w_ref, pf_ref, Wfc1_ref, bfc1_ref, Wfc2_ref, bfc2_ref,
                  Wfc3_ref, bfc3_ref, WconvT_ref, bconv_ref,
                  imgn_ref, att_ref, s1_ref, s2_ref):
    v4 = v4_ref[...]
    w8 = w_ref[...]
    img = (v4[0, :, :_Ci] * w8[0][:, None] + v4[0, :, _Ci:] * w8[1][:, None]
           + v4[1, :, :_Ci] * w8[2][:, None] + v4[1, :, _Ci:] * w8[3][:, None]
           + v4[2, :, :_Ci] * w8[4][:, None] + v4[2, :, _Ci:] * w8[5][:, None]
           + v4[3, :, :_Ci] * w8[6][:, None] + v4[3, :, _Ci:] * w8[7][:, None])
    pf = pf_ref[...]
    ri = jnp.dot(img, Wfc1_ref[...], preferred_element_type=jnp.float32) + bfc1_ref[...]
    rp = jnp.dot(pf, Wfc2_ref[...], preferred_element_type=jnp.float32) + bfc2_ref[...]
    t = jnp.tanh(ri + rp)
    att = jax.nn.sigmoid(
        jnp.dot(t, Wfc3_ref[...], preferred_element_type=jnp.float32) + bfc3_ref[...])
    imgn = jnp.dot(img, WconvT_ref[...], preferred_element_type=jnp.float32) + bconv_ref[...]
    imgn_ref[...] = imgn
    att_ref[...] = att

    @pl.when(pl.program_id(0) == 0)
    def _():
        s1_ref[...] = jnp.zeros_like(s1_ref)
        s2_ref[...] = jnp.zeros_like(s2_ref)

    s1_ref[...] += jnp.sum(imgn, axis=0, keepdims=True)
    s2_ref[...] += jnp.sum(imgn * imgn, axis=0, keepdims=True)


def _dense_a(v4, wgt, pf, Wfc1, bfc1, Wfc2, bfc2, Wfc3, bfc3, WconvT, bconv):
    grid = _M // _TM
    return pl.pallas_call(
        _dense_a_body,
        grid=(grid,),
        in_specs=[
            pl.BlockSpec((4, _TM, 128), lambda i: (0, i, 0)),
            pl.BlockSpec((8, _TM), lambda i: (0, i)),
            pl.BlockSpec((_TM, _Cp), lambda i: (i, 0)),
            pl.BlockSpec((_Ci, 24), lambda i: (0, 0)),
            pl.BlockSpec((1, 24), lambda i: (0, 0)),
            pl.BlockSpec((_Cp, 24), lambda i: (0, 0)),
            pl.BlockSpec((1, 24), lambda i: (0, 0)),
            pl.BlockSpec((24, 1), lambda i: (0, 0)),
            pl.BlockSpec((1, 1), lambda i: (0, 0)),
            pl.BlockSpec((_Ci, _Cp), lambda i: (0, 0)),
            pl.BlockSpec((1, _Cp), lambda i: (0, 0)),
        ],
        out_specs=[
            pl.BlockSpec((_TM, _Cp), lambda i: (i, 0)),
            pl.BlockSpec((_TM, 1), lambda i: (i, 0)),
            pl.BlockSpec((1, _Cp), lambda i: (0, 0)),
            pl.BlockSpec((1, _Cp), lambda i: (0, 0)),
        ],
        out_shape=[
            jax.ShapeDtypeStruct((_M, _Cp), jnp.float32),
            jax.ShapeDtypeStruct((_M, 1), jnp.float32),
            jax.ShapeDtypeStruct((1, _Cp), jnp.float32),
            jax.ShapeDtypeStruct((1, _Cp), jnp.float32),
        ],
        compiler_params=pltpu.CompilerParams(
            dimension_semantics=("arbitrary",)),
    )(v4, wgt, pf, Wfc1, bfc1, Wfc2, bfc2, Wfc3, bfc3, WconvT, bconv)


def _dense_b_body(imgn_ref, att_ref, pf_ref, s1_ref, s2_ref, g1_ref, be1_ref,
                  WfpT_ref, WfiT_ref, bfuse_ref, h_ref, t1_ref, t2_ref):
    s1 = s1_ref[...]
    s2 = s2_ref[...]
    mean = s1 * (1.0 / _M)
    var = s2 * (1.0 / _M) - mean * mean
    scale = g1_ref[...] * jax.lax.rsqrt(var + _EPS)
    shift = be1_ref[...] - mean * scale
    img_out = jnp.maximum(imgn_ref[...] * scale + shift, 0.0) * att_ref[...]
    h = (jnp.dot(pf_ref[...], WfpT_ref[...], preferred_element_type=jnp.float32)
         + jnp.dot(img_out, WfiT_ref[...], preferred_element_type=jnp.float32)
         + bfuse_ref[...])
    h_ref[...] = h

    @pl.when(pl.program_id(0) == 0)
    def _():
        t1_ref[...] = jnp.zeros_like(t1_ref)
        t2_ref[...] = jnp.zeros_like(t2_ref)

    t1_ref[...] += jnp.sum(h, axis=0, keepdims=True)
    t2_ref[...] += jnp.sum(h * h, axis=0, keepdims=True)


def _dense_b(imgn, att, pf, s1, s2, g1, be1, WfpT, WfiT, bfuse):
    grid = _M // _TM
    return pl.pallas_call(
        _dense_b_body,
        grid=(grid,),
        in_specs=[
            pl.BlockSpec((_TM, _Cp), lambda i: (i, 0)),
            pl.BlockSpec((_TM, 1), lambda i: (i, 0)),
            pl.BlockSpec((_TM, _Cp), lambda i: (i, 0)),
            pl.BlockSpec((1, _Cp), lambda i: (0, 0)),
            pl.BlockSpec((1, _Cp), lambda i: (0, 0)),
            pl.BlockSpec((1, _Cp), lambda i: (0, 0)),
            pl.BlockSpec((1, _Cp), lambda i: (0, 0)),
            pl.BlockSpec((_Cp, _Cp), lambda i: (0, 0)),
            pl.BlockSpec((_Cp, _Cp), lambda i: (0, 0)),
            pl.BlockSpec((1, _Cp), lambda i: (0, 0)),
        ],
        out_specs=[
            pl.BlockSpec((_TM, _Cp), lambda i: (i, 0)),
            pl.BlockSpec((1, _Cp), lambda i: (0, 0)),
            pl.BlockSpec((1, _Cp), lambda i: (0, 0)),
        ],
        out_shape=[
            jax.ShapeDtypeStruct((_M, _Cp), jnp.float32),
            jax.ShapeDtypeStruct((1, _Cp), jnp.float32),
            jax.ShapeDtypeStruct((1, _Cp), jnp.float32),
        ],
        compiler_params=pltpu.CompilerParams(
            dimension_semantics=("arbitrary",)),
    )(imgn, att, pf, s1, s2, g1, be1, WfpT, WfiT, bfuse)


def _dense_c_body(h_ref, t1_ref, t2_ref, g2_ref, be2_ref, o_ref):
    t1 = t1_ref[...]
    t2 = t2_ref[...]
    mean = t1 * (1.0 / _M)
    var = t2 * (1.0 / _M) - mean * mean
    scale = g2_ref[...] * jax.lax.rsqrt(var + _EPS)
    shift = be2_ref[...] - mean * scale
    o = jnp.maximum(h_ref[...] * scale + shift, 0.0)
    o_ref[0] = o.T


def _dense_c(h, t1, t2, g2, be2):
    grid = _M // _TM
    return pl.pallas_call(
        _dense_c_body,
        grid=(grid,),
        in_specs=[
            pl.BlockSpec((_TM, _Cp), lambda i: (i, 0)),
            pl.BlockSpec((1, _Cp), lambda i: (0, 0)),
            pl.BlockSpec((1, _Cp), lambda i: (0, 0)),
            pl.BlockSpec((1, _Cp), lambda i: (0, 0)),
            pl.BlockSpec((1, _Cp), lambda i: (0, 0)),
        ],
        out_specs=pl.BlockSpec((1, _Cp, _TM),
                               lambda i: (i // (_N // _TM), 0, i % (_N // _TM))),
        out_shape=jax.ShapeDtypeStruct((_B, _Cp, _N), jnp.float32),
        compiler_params=pltpu.CompilerParams(
            dimension_semantics=("arbitrary",)),
    )(h, t1, t2, g2, be2)


@jax.jit
def kernel(point_features, image, xy, Wfc1, bfc1, Wfc2, bfc2, Wfc3, bfc3,
           Wconv, bconv, g1, be1, Wfuse, bfuse, g2, be2):
    table = image.transpose(0, 2, 3, 1).reshape(_B * _H * _W // 2, 2 * _Ci)
    pf = point_features.transpose(0, 2, 1).reshape(_M, _Cp)
    xn = xy[..., 0].reshape(_M)
    yn = xy[..., 1].reshape(_M)

    idx, wgt = _compute_idx(xn, yn)
    gathered = _sc_gather(table, idx.reshape(4 * _M))
    v4 = gathered.reshape(4, _M, 128)
    wgt_m = wgt.reshape(8, _M)

    imgn, att, s1, s2 = _dense_a(
        v4, wgt_m, pf,
        Wfc1, bfc1.reshape(1, 24), Wfc2, bfc2.reshape(1, 24),
        Wfc3, bfc3.reshape(1, 1),
        Wconv.T, bconv.reshape(1, _Cp))

    h, t1, t2 = _dense_b(
        imgn, att, pf, s1, s2,
        g1.reshape(1, _Cp), be1.reshape(1, _Cp),
        Wfuse[:, :_Cp].T, Wfuse[:, _Cp:].T, bfuse.reshape(1, _Cp))

    return _dense_c(h, t1, t2, g2.reshape(1, _Cp), be2.reshape(1, _Cp))


# speedup vs baseline: 1.2815x; 1.1088x over previous
import functools

import jax
import jax.numpy as jnp
from jax import lax
from jax.experimental import pallas as pl
from jax.experimental.pallas import tpu as pltpu
from jax.experimental.pallas import tpu_sc as plsc

_B, _N, _Ci, _Cp, _H, _W = 2, 16384, 64, 96, 192, 640
_M = _B * _N
_TM = 4096
_CH = 128
_EPS = 1e-5


def _idx_body(xn_ref, yn_ref, idx_ref, wgt_ref):
    xn = xn_ref[...]
    yn = yn_ref[...]
    x = (xn + 1.0) * (_W / 2.0) - 0.5
    y = (yn + 1.0) * (_H / 2.0) - 0.5
    x0 = jnp.floor(x)
    y0 = jnp.floor(y)
    wx1 = x - x0
    wx0 = 1.0 - wx1
    wy1 = y - y0
    wy0 = 1.0 - wy1
    rows = xn.shape[0]
    boff = (lax.broadcasted_iota(jnp.int32, (rows, 128), 0) // (_N // 128)) * (_H * _W)

    def corner(c, xi, yi, wgt):
        valid = ((xi >= 0) & (xi <= _W - 1) & (yi >= 0) & (yi <= _H - 1))
        xi_c = jnp.clip(xi, 0.0, _W - 1).astype(jnp.int32)
        yi_c = jnp.clip(yi, 0.0, _H - 1).astype(jnp.int32)
        pix = yi_c * _W + xi_c + boff
        w = wgt * valid.astype(jnp.float32)
        par = (pix & 1).astype(jnp.float32)
        idx_ref[c] = pix >> 1
        wgt_ref[2 * c] = w * (1.0 - par)
        wgt_ref[2 * c + 1] = w * par

    corner(0, x0, y0, wx0 * wy0)
    corner(1, x0 + 1.0, y0, wx1 * wy0)
    corner(2, x0, y0 + 1.0, wx0 * wy1)
    corner(3, x0 + 1.0, y0 + 1.0, wx1 * wy1)


def _compute_idx(xn, yn):
    rows = _M // 128
    return pl.pallas_call(
        _idx_body,
        out_shape=(
            jax.ShapeDtypeStruct((4, rows, 128), jnp.int32),
            jax.ShapeDtypeStruct((8, rows, 128), jnp.float32),
        ),
    )(xn.reshape(rows, 128), yn.reshape(rows, 128))


def _sc_gather(table, idx):
    total = 4 * _M
    info = plsc.get_sparse_core_info()
    nw = info.num_cores * info.num_subcores
    per_w = total // nw
    n_chunks = per_w // _CH
    mesh = plsc.VectorSubcoreMesh(core_axis_name="c", subcore_axis_name="s")

    nb = 4
    n_groups = n_chunks // nb

    @functools.partial(
        pl.kernel,
        mesh=mesh,
        out_type=jax.ShapeDtypeStruct((total, 128), jnp.float32),
        scratch_types=(
            [pltpu.VMEM((per_w,), jnp.int32)]
            + [pltpu.VMEM((_CH, 128), jnp.float32) for _ in range(nb)]
            + [pltpu.SemaphoreType.DMA, pltpu.SemaphoreType.DMA]
        ),
    )
    def k(table_hbm, idx_hbm, out_hbm, idx_v, b0, b1, b2, b3, gsem, ssem):
        bufs = (b0, b1, b2, b3)
        wid = lax.axis_index("s") * info.num_cores + lax.axis_index("c")
        base = wid * per_w
        pltpu.sync_copy(idx_hbm.at[pl.ds(base, per_w)], idx_v)

        def body(g, carry):
            off0 = g * (nb * _CH)
            copies = []
            for b in range(nb):
                idx_slice = idx_v.at[pl.ds(off0 + b * _CH, _CH)]
                copies.append(pltpu.async_copy(table_hbm.at[idx_slice], bufs[b], gsem))
            for c in copies:
                c.wait()
            stores = []
            for b in range(nb):
                dst = out_hbm.at[pl.ds(base + off0 + b * _CH, _CH)]
                stores.append(pltpu.async_copy(bufs[b], dst, ssem))
            for s in stores:
                s.wait()
            return carry

        lax.fori_loop(0, n_groups, body, 0)

    return k(table, idx)


def _dense_a_body(v4_ref, w_ref, pf_ref, Wfc1_ref, bfc1_ref, Wfc2_ref, bfc2_ref,
                  Wfc3_ref, bfc3_ref, WconvT_ref, bconv_ref,
                  imgn_ref, att_ref, s1_ref, s2_ref):
    v4 = v4_ref[...]
    w8 = w_ref[...]
    img = (v4[0, :, :_Ci] * w8[0][:, None] + v4[0, :, _Ci:] * w8[1][:, None]
           + v4[1, :, :_Ci] * w8[2][:, None] + v4[1, :, _Ci:] * w8[3][:, None]
           + v4[2, :, :_Ci] * w8[4][:, None] + v4[2, :, _Ci:] * w8[5][:, None]
           + v4[3, :, :_Ci] * w8[6][:, None] + v4[3, :, _Ci:] * w8[7][:, None])
    pf = pf_ref[0].T
    ri = jnp.dot(img, Wfc1_ref[...], preferred_element_type=jnp.float32) + bfc1_ref[...]
    rp = jnp.dot(pf, Wfc2_ref[...], preferred_element_type=jnp.float32) + bfc2_ref[...]
    t = jnp.tanh(ri + rp)
    att = jax.nn.sigmoid(
        jnp.dot(t, Wfc3_ref[...], preferred_element_type=jnp.float32) + bfc3_ref[...])
    imgn = jnp.dot(img, WconvT_ref[...], preferred_element_type=jnp.float32) + bconv_ref[...]
    imgn_ref[...] = imgn
    att_ref[...] = att

    @pl.when(pl.program_id(0) == 0)
    def _():
        s1_ref[...] = jnp.zeros_like(s1_ref)
        s2_ref[...] = jnp.zeros_like(s2_ref)

    s1_ref[...] += jnp.sum(imgn, axis=0, keepdims=True)
    s2_ref[...] += jnp.sum(imgn * imgn, axis=0, keepdims=True)


def _dense_a(v4, wgt, pf, Wfc1, bfc1, Wfc2, bfc2, Wfc3, bfc3, WconvT, bconv):
    grid = _M // _TM
    return pl.pallas_call(
        _dense_a_body,
        grid=(grid,),
        in_specs=[
            pl.BlockSpec((4, _TM, 128), lambda i: (0, i, 0)),
            pl.BlockSpec((8, _TM), lambda i: (0, i)),
            pl.BlockSpec((1, _Cp, _TM),
                         lambda i: (i // (_N // _TM), 0, i % (_N // _TM))),
            pl.BlockSpec((_Ci, 24), lambda i: (0, 0)),
            pl.BlockSpec((1, 24), lambda i: (0, 0)),
            pl.BlockSpec((_Cp, 24), lambda i: (0, 0)),
            pl.BlockSpec((1, 24), lambda i: (0, 0)),
            pl.BlockSpec((24, 1), lambda i: (0, 0)),
            pl.BlockSpec((1, 1), lambda i: (0, 0)),
            pl.BlockSpec((_Ci, _Cp), lambda i: (0, 0)),
            pl.BlockSpec((1, _Cp), lambda i: (0, 0)),
        ],
        out_specs=[
            pl.BlockSpec((_TM, _Cp), lambda i: (i, 0)),
            pl.BlockSpec((_TM, 1), lambda i: (i, 0)),
            pl.BlockSpec((1, _Cp), lambda i: (0, 0)),
            pl.BlockSpec((1, _Cp), lambda i: (0, 0)),
        ],
        out_shape=[
            jax.ShapeDtypeStruct((_M, _Cp), jnp.float32),
            jax.ShapeDtypeStruct((_M, 1), jnp.float32),
            jax.ShapeDtypeStruct((1, _Cp), jnp.float32),
            jax.ShapeDtypeStruct((1, _Cp), jnp.float32),
        ],
        compiler_params=pltpu.CompilerParams(
            dimension_semantics=("arbitrary",)),
    )(v4, wgt, pf, Wfc1, bfc1, Wfc2, bfc2, Wfc3, bfc3, WconvT, bconv)


def _dense_b_body(imgn_ref, att_ref, pf_ref, s1_ref, s2_ref, g1_ref, be1_ref,
                  WfpT_ref, WfiT_ref, bfuse_ref, h_ref, t1_ref, t2_ref):
    s1 = s1_ref[...]
    s2 = s2_ref[...]
    mean = s1 * (1.0 / _M)
    var = s2 * (1.0 / _M) - mean * mean
    scale = g1_ref[...] * jax.lax.rsqrt(var + _EPS)
    shift = be1_ref[...] - mean * scale
    img_out = jnp.maximum(imgn_ref[...] * scale + shift, 0.0) * att_ref[...]
    h = (jnp.dot(pf_ref[0].T, WfpT_ref[...], preferred_element_type=jnp.float32)
         + jnp.dot(img_out, WfiT_ref[...], preferred_element_type=jnp.float32)
         + bfuse_ref[...])
    h_ref[...] = h

    @pl.when(pl.program_id(0) == 0)
    def _():
        t1_ref[...] = jnp.zeros_like(t1_ref)
        t2_ref[...] = jnp.zeros_like(t2_ref)

    t1_ref[...] += jnp.sum(h, axis=0, keepdims=True)
    t2_ref[...] += jnp.sum(h * h, axis=0, keepdims=True)


def _dense_b(imgn, att, pf, s1, s2, g1, be1, WfpT, WfiT, bfuse):
    grid = _M // _TM
    return pl.pallas_call(
        _dense_b_body,
        grid=(grid,),
        in_specs=[
            pl.BlockSpec((_TM, _Cp), lambda i: (i, 0)),
            pl.BlockSpec((_TM, 1), lambda i: (i, 0)),
            pl.BlockSpec((1, _Cp, _TM),
                         lambda i: (i // (_N // _TM), 0, i % (_N // _TM))),
            pl.BlockSpec((1, _Cp), lambda i: (0, 0)),
            pl.BlockSpec((1, _Cp), lambda i: (0, 0)),
            pl.BlockSpec((1, _Cp), lambda i: (0, 0)),
            pl.BlockSpec((1, _Cp), lambda i: (0, 0)),
            pl.BlockSpec((_Cp, _Cp), lambda i: (0, 0)),
            pl.BlockSpec((_Cp, _Cp), lambda i: (0, 0)),
            pl.BlockSpec((1, _Cp), lambda i: (0, 0)),
        ],
        out_specs=[
            pl.BlockSpec((_TM, _Cp), lambda i: (i, 0)),
            pl.BlockSpec((1, _Cp), lambda i: (0, 0)),
            pl.BlockSpec((1, _Cp), lambda i: (0, 0)),
        ],
        out_shape=[
            jax.ShapeDtypeStruct((_M, _Cp), jnp.float32),
            jax.ShapeDtypeStruct((1, _Cp), jnp.float32),
            jax.ShapeDtypeStruct((1, _Cp), jnp.float32),
        ],
        compiler_params=pltpu.CompilerParams(
            dimension_semantics=("arbitrary",)),
    )(imgn, att, pf, s1, s2, g1, be1, WfpT, WfiT, bfuse)


def _dense_c_body(h_ref, t1_ref, t2_ref, g2_ref, be2_ref, o_ref):
    t1 = t1_ref[...]
    t2 = t2_ref[...]
    mean = t1 * (1.0 / _M)
    var = t2 * (1.0 / _M) - mean * mean
    scale = g2_ref[...] * jax.lax.rsqrt(var + _EPS)
    shift = be2_ref[...] - mean * scale
    o = jnp.maximum(h_ref[...] * scale + shift, 0.0)
    o_ref[0] = o.T


def _dense_c(h, t1, t2, g2, be2):
    grid = _M // _TM
    return pl.pallas_call(
        _dense_c_body,
        grid=(grid,),
        in_specs=[
            pl.BlockSpec((_TM, _Cp), lambda i: (i, 0)),
            pl.BlockSpec((1, _Cp), lambda i: (0, 0)),
            pl.BlockSpec((1, _Cp), lambda i: (0, 0)),
            pl.BlockSpec((1, _Cp), lambda i: (0, 0)),
            pl.BlockSpec((1, _Cp), lambda i: (0, 0)),
        ],
        out_specs=pl.BlockSpec((1, _Cp, _TM),
                               lambda i: (i // (_N // _TM), 0, i % (_N // _TM))),
        out_shape=jax.ShapeDtypeStruct((_B, _Cp, _N), jnp.float32),
        compiler_params=pltpu.CompilerParams(
            dimension_semantics=("arbitrary",)),
    )(h, t1, t2, g2, be2)


@jax.jit
def kernel(point_features, image, xy, Wfc1, bfc1, Wfc2, bfc2, Wfc3, bfc3,
           Wconv, bconv, g1, be1, Wfuse, bfuse, g2, be2):
    table = image.transpose(0, 2, 3, 1).reshape(_B * _H * _W // 2, 2 * _Ci)
    pf = point_features
    xn = xy[..., 0].reshape(_M)
    yn = xy[..., 1].reshape(_M)

    idx, wgt = _compute_idx(xn, yn)
    gathered = _sc_gather(table, idx.reshape(4 * _M))
    v4 = gathered.reshape(4, _M, 128)
    wgt_m = wgt.reshape(8, _M)

    imgn, att, s1, s2 = _dense_a(
        v4, wgt_m, pf,
        Wfc1, bfc1.reshape(1, 24), Wfc2, bfc2.reshape(1, 24),
        Wfc3, bfc3.reshape(1, 1),
        Wconv.T, bconv.reshape(1, _Cp))

    h, t1, t2 = _dense_b(
        imgn, att, pf, s1, s2,
        g1.reshape(1, _Cp), be1.reshape(1, _Cp),
        Wfuse[:, :_Cp].T, Wfuse[:, _Cp:].T, bfuse.reshape(1, _Cp))

    return _dense_c(h, t1, t2, g2.reshape(1, _Cp), be2.reshape(1, _Cp))
